# baseline (speedup 1.0000x reference)
import functools

import jax
import jax.numpy as jnp
from jax import lax
from jax.experimental import pallas as pl
from jax.experimental.pallas import tpu as pltpu
from jax.experimental.pallas import tpu_sc as plsc

N = 10000
E = 320000
DIN = 128
D = 256
NC = 2
NS = 16
HN = N // NC

EPAD = 327680
SCN = EPAD // NS
SB = 2048
NSB = SCN // SB
CAP = 20992
CROWS = 168
CAPH = CROWS * 128
CAPB = CAPH // 64
SINK = HN

CH = 64
SCW = 128
NG = 8
EPI = NG * SCW

ACC = 5120
ZR = ACC // NS
OW = 312
TAILO = NS * OW
TAILN = HN - TAILO

BM = 1000
G = N // BM

_sc_mesh = plsc.VectorSubcoreMesh(core_axis_name="c", subcore_axis_name="s")


def _bucket_edges(src, dst):
    m0 = dst < HN
    m0i = m0.astype(jnp.int32)
    c0 = jnp.cumsum(m0i)
    c1 = jnp.cumsum(1 - m0i)
    total0 = c0[-1]
    r = jnp.where(m0, c0, c1) - 1
    core = jnp.where(m0, 0, 1)
    gpos = (core * NS + r % NS) * CAPH + r // NS
    sl = jnp.zeros((NC * NS * CAPH,), jnp.int32)
    sl = sl.at[gpos].set(src, mode="promise_in_bounds", unique_indices=True)
    dl = jnp.full((NC * NS * CAPH,), SINK, jnp.int32)
    dl = dl.at[gpos].set(dst - core * HN, mode="promise_in_bounds",
                         unique_indices=True)
    t16 = jnp.arange(NS, dtype=jnp.int32)
    cnt = jnp.concatenate([(total0 - t16 + NS - 1) // NS,
                           ((E - total0) - t16 + NS - 1) // NS])
    cnt = jnp.broadcast_to(cnt[:, None], (NC * NS, 16)).reshape(-1)
    return (sl.reshape(NC * NS, CAPB, 64),
            dl.reshape(NC * NS, CAPH // SCW, SCW),
            cnt.astype(jnp.int32))


def _sc_segment_sum(h, sl4, dl4, cnt, zinit):
    @functools.partial(
        pl.kernel,
        out_type=jax.ShapeDtypeStruct((N, 2, 128), jnp.float32),
        mesh=_sc_mesh,
        scratch_types=[
            pltpu.VMEM((EPI // CH, CH), jnp.int32),
            pltpu.VMEM((NG, SCW), jnp.int32),
            pltpu.VMEM((SCW, 2, 128), jnp.float32),
            pltpu.VMEM((16,), jnp.int32),
            pltpu.VMEM_SHARED((ACC, 2, 128), jnp.float32),
            [pltpu.SemaphoreType.DMA] * 2,
            pltpu.SemaphoreType.DMA,
        ],
    )
    def run(h_hbm, sl_hbm, dl_hbm, cnt_hbm, z_hbm, out_hbm,
            s_v, d_v, rows, cnt_v, acc, gsem, ssem):
        cid = lax.axis_index("c")
        sid = lax.axis_index("s")
        w = cid * NS + sid
        pltpu.sync_copy(z_hbm, acc.at[pl.ds(sid * ZR, ZR)])
        pltpu.sync_copy(cnt_hbm.at[pl.ds(w * 16, 16)], cnt_v)
        plsc.subcore_barrier()
        nit = lax.div(cnt_v[...][0] + (EPI - 1), EPI)

        def it(jb, carry):
            pltpu.sync_copy(sl_hbm.at[w, pl.ds(jb * (EPI // CH), EPI // CH)],
                            s_v)
            pltpu.sync_copy(dl_hbm.at[w, pl.ds(jb * NG, NG)], d_v)
            for g in range(NG):
                g0 = pltpu.async_copy(h_hbm.at[s_v.at[2 * g]],
                                      rows.at[pl.ds(0, CH)], gsem[0])
                g1 = pltpu.async_copy(h_hbm.at[s_v.at[2 * g + 1]],
                                      rows.at[pl.ds(CH, CH)], gsem[1])
                g0.wait()
                g1.wait()
                sc = pltpu.async_copy(rows, acc.at[d_v.at[g]], ssem,
                                      add=True)
                sc.wait()
            return carry

        lax.fori_loop(0, nit, it, 0)
        plsc.subcore_barrier()
        pltpu.sync_copy(
            acc.at[pl.ds(sid * OW, OW)],
            out_hbm.at[pl.ds(cid * HN + sid * OW, OW)],
        )

        @pl.when(sid == NS - 1)
        def _():
            pltpu.sync_copy(
                acc.at[pl.ds(TAILO, TAILN)],
                out_hbm.at[pl.ds(cid * HN + TAILO, TAILN)],
            )

    out = run(h.reshape(N, 2, 128), sl4, dl4, cnt,
              zinit.reshape(ZR, 2, 128))
    return out.reshape(N, D)


def _relu(v):
    return jnp.maximum(v, 0.0)


def _dot(a, b):
    return jnp.dot(a, b, preferred_element_type=jnp.float32)


def _enc_body(x_ref, w_ref, b_ref, o_ref):
    o_ref[...] = _relu(_dot(x_ref[...], w_ref[...]) + b_ref[...])


def _gin_mid_body(h_ref, a_ref, w1_ref, b1_ref, w2_ref, b2_ref, o_ref):
    z = h_ref[...] + a_ref[...]
    t = _relu(_dot(z, w1_ref[...]) + b1_ref[...])
    o_ref[...] = _relu(_dot(t, w2_ref[...]) + b2_ref[...])


def _gin_final_body(h_ref, a_ref, w1_ref, b1_ref, w2_ref, b2_ref, o_ref, s_ref):
    z = h_ref[...] + a_ref[...]
    t = _relu(_dot(z, w1_ref[...]) + b1_ref[...])
    u = _dot(t, w2_ref[...]) + b2_ref[...]
    o_ref[...] = u
    part = jnp.sum(u, axis=0, keepdims=True)
    i = pl.program_id(0)

    @pl.when(i == 0)
    def _():
        s_ref[...] = part

    @pl.when(i > 0)
    def _():
        s_ref[...] = s_ref[...] + part

    @pl.when(i == G - 1)
    def _():
        s_ref[...] = s_ref[...] * (1.0 / N)


_h_spec = pl.BlockSpec((BM, D), lambda i: (i, 0))
_w_spec = pl.BlockSpec((D, D), lambda i: (0, 0))
_b_spec = pl.BlockSpec((1, D), lambda i: (0, 0))


def _encoder(x, w, b):
    return pl.pallas_call(
        _enc_body,
        grid=(G,),
        in_specs=[
            pl.BlockSpec((BM, DIN), lambda i: (i, 0)),
            pl.BlockSpec((DIN, D), lambda i: (0, 0)),
            _b_spec,
        ],
        out_specs=_h_spec,
        out_shape=jax.ShapeDtypeStruct((N, D), jnp.float32),
    )(x, w, b)


def _gin_mid(h, agg, w1, b1, w2, b2):
    return pl.pallas_call(
        _gin_mid_body,
        grid=(G,),
        in_specs=[_h_spec, _h_spec, _w_spec, _b_spec, _w_spec, _b_spec],
        out_specs=_h_spec,
        out_shape=jax.ShapeDtypeStruct((N, D), jnp.float32),
    )(h, agg, w1, b1, w2, b2)


def _gin_final(h, agg, w1, b1, w2, b2):
    return pl.pallas_call(
        _gin_final_body,
        grid=(G,),
        in_specs=[_h_spec, _h_spec, _w_spec, _b_spec, _w_spec, _b_spec],
        out_specs=[
            _h_spec,
            pl.BlockSpec((1, D), lambda i: (0, 0)),
        ],
        out_shape=[
            jax.ShapeDtypeStruct((N, D), jnp.float32),
            jax.ShapeDtypeStruct((1, D), jnp.float32),
        ],
    )(h, agg, w1, b1, w2, b2)


def kernel(x, edge_index, W_enc, b_enc, gin_W1, gin_b1, gin_W2, gin_b2):
    sl4, dl4, cnt = _bucket_edges(edge_index[0], edge_index[1])
    zinit = jnp.zeros((ZR, D), jnp.float32)

    b_enc2 = b_enc.reshape(1, D)
    b1 = gin_b1.reshape(-1, 1, D)
    b2 = gin_b2.reshape(-1, 1, D)

    h = _encoder(x, W_enc, b_enc2)
    h0 = h

    for l in range(2):
        agg = _sc_segment_sum(h, sl4, dl4, cnt, zinit)
        h = _gin_mid(h, agg, gin_W1[l], b1[l], gin_W2[l], b2[l])

    agg = _sc_segment_sum(h, sl4, dl4, cnt, zinit)
    h, s = _gin_final(h, agg, gin_W1[2], b1[2], gin_W2[2], b2[2])
    return (h, s[0], h0)


# speedup vs baseline: 2.2014x; 2.2014x over previous
import functools

import jax
import jax.numpy as jnp
from jax import lax
from jax.experimental import pallas as pl
from jax.experimental.pallas import tpu as pltpu
from jax.experimental.pallas import tpu_sc as plsc

N = 10000
E = 320000
DIN = 128
D = 256
HALF = 128
NC = 2
NS = 16
CH = 128
NCH = 160
KB = 16
NBLK = NCH // KB
EPAD = NS * NCH * CH
ACC = 10112
ZR = ACC // NS
OPT = 624
TAILO = NS * OPT
TAILN = N - TAILO

BM = 1000
G = N // BM


def _sc_segment_sum(hflat, src3, dst3, zinit):
    @functools.partial(
        pl.kernel,
        out_type=jax.ShapeDtypeStruct((NC * N, HALF), jnp.float32),
        mesh=plsc.VectorSubcoreMesh(core_axis_name="c", subcore_axis_name="s"),
        scratch_types=[
            pltpu.VMEM((KB, CH), jnp.int32),
            pltpu.VMEM((KB, CH), jnp.int32),
            [pltpu.VMEM((CH, HALF), jnp.float32)] * 2,
            pltpu.VMEM_SHARED((ACC, HALF), jnp.float32),
            [pltpu.SemaphoreType.DMA] * 2,
            [pltpu.SemaphoreType.DMA] * 2,
        ],
    )
    def run(h_hbm, s_hbm, d_hbm, z_hbm, out_hbm, s_v, d_v, rows,
            acc, gsem, ssem):
        cid = lax.axis_index("c")
        sid = lax.axis_index("s")
        pltpu.sync_copy(z_hbm, acc.at[pl.ds(sid * ZR, ZR)])
        plsc.subcore_barrier()

        def blk(b, carry):
            pltpu.sync_copy(s_hbm.at[cid * NS + sid, pl.ds(b * KB, KB)], s_v)
            pltpu.sync_copy(d_hbm.at[sid, pl.ds(b * KB, KB)], d_v)
            g = pltpu.async_copy(h_hbm.at[s_v.at[0]], rows[0], gsem[0])
            sc = [None, None]
            for j in range(KB):
                bj = j % 2
                nb = (j + 1) % 2
                if j + 1 < KB:
                    if sc[nb] is not None:
                        sc[nb].wait()
                    gn = pltpu.async_copy(h_hbm.at[s_v.at[j + 1]], rows[nb],
                                          gsem[nb])
                g.wait()
                sc[bj] = pltpu.async_copy(rows[bj], acc.at[d_v.at[j]],
                                          ssem[bj], add=True)
                if j + 1 < KB:
                    g = gn
            sc[0].wait()
            sc[1].wait()
            return carry

        lax.fori_loop(0, NBLK, blk, 0)
        plsc.subcore_barrier()
        pltpu.sync_copy(
            acc.at[pl.ds(sid * OPT, OPT)],
            out_hbm.at[pl.ds(cid * N + sid * OPT, OPT)],
        )

        @pl.when(sid == NS - 1)
        def _():
            pltpu.sync_copy(
                acc.at[pl.ds(TAILO, TAILN)],
                out_hbm.at[pl.ds(cid * N + TAILO, TAILN)],
            )

    return run(hflat, src3, dst3, zinit)


def _relu(v):
    return jnp.maximum(v, 0.0)


def _dot(a, b):
    return jnp.dot(a, b, preferred_element_type=jnp.float32)


def _enc_body(x_ref, w_ref, b_ref, o_ref):
    h = _relu(_dot(x_ref[...], w_ref[...]) + b_ref[...])
    o_ref[0] = h[:, :HALF]
    o_ref[1] = h[:, HALF:]


def _gin_mid_body(h_ref, a_ref, w1_ref, b1_ref, w2_ref, b2_ref, o_ref):
    z0 = h_ref[0] + a_ref[0]
    z1 = h_ref[1] + a_ref[1]
    w1 = w1_ref[...]
    t = _relu(_dot(z0, w1[:HALF]) + _dot(z1, w1[HALF:]) + b1_ref[...])
    u = _relu(_dot(t, w2_ref[...]) + b2_ref[...])
    o_ref[0] = u[:, :HALF]
    o_ref[1] = u[:, HALF:]


def _gin_final_body(h_ref, a_ref, w1_ref, b1_ref, w2_ref, b2_ref, o_ref, s_ref):
    z0 = h_ref[0] + a_ref[0]
    z1 = h_ref[1] + a_ref[1]
    w1 = w1_ref[...]
    t = _relu(_dot(z0, w1[:HALF]) + _dot(z1, w1[HALF:]) + b1_ref[...])
    u = _dot(t, w2_ref[...]) + b2_ref[...]
    o_ref[...] = u
    part = jnp.sum(u, axis=0, keepdims=True)
    i = pl.program_id(0)

    @pl.when(i == 0)
    def _():
        s_ref[...] = part

    @pl.when(i > 0)
    def _():
        s_ref[...] = s_ref[...] + part

    @pl.when(i == G - 1)
    def _():
        s_ref[...] = s_ref[...] * (1.0 / N)


_parts_spec = pl.BlockSpec((NC, BM, HALF), lambda i: (0, i, 0))
_w_spec = pl.BlockSpec((D, D), lambda i: (0, 0))
_b_spec = pl.BlockSpec((1, D), lambda i: (0, 0))


def _encoder(x, w, b):
    return pl.pallas_call(
        _enc_body,
        grid=(G,),
        in_specs=[
            pl.BlockSpec((BM, DIN), lambda i: (i, 0)),
            pl.BlockSpec((DIN, D), lambda i: (0, 0)),
            _b_spec,
        ],
        out_specs=_parts_spec,
        out_shape=jax.ShapeDtypeStruct((NC, N, HALF), jnp.float32),
    )(x, w, b)


def _gin_mid(hp, agg, w1, b1, w2, b2):
    return pl.pallas_call(
        _gin_mid_body,
        grid=(G,),
        in_specs=[_parts_spec, _parts_spec, _w_spec, _b_spec, _w_spec, _b_spec],
        out_specs=_parts_spec,
        out_shape=jax.ShapeDtypeStruct((NC, N, HALF), jnp.float32),
    )(hp, agg, w1, b1, w2, b2)


def _gin_final(hp, agg, w1, b1, w2, b2):
    return pl.pallas_call(
        _gin_final_body,
        grid=(G,),
        in_specs=[_parts_spec, _parts_spec, _w_spec, _b_spec, _w_spec, _b_spec],
        out_specs=[
            pl.BlockSpec((BM, D), lambda i: (i, 0)),
            pl.BlockSpec((1, D), lambda i: (0, 0)),
        ],
        out_shape=[
            jax.ShapeDtypeStruct((N, D), jnp.float32),
            jax.ShapeDtypeStruct((1, D), jnp.float32),
        ],
    )(hp, agg, w1, b1, w2, b2)


def kernel(x, edge_index, W_enc, b_enc, gin_W1, gin_b1, gin_W2, gin_b2):
    src = edge_index[0]
    dst = edge_index[1]
    pad = EPAD - E
    src_p = jnp.concatenate([src, jnp.zeros((pad,), jnp.int32)])
    dst_p = jnp.concatenate([dst, jnp.full((pad,), N, jnp.int32)])
    src_t = src_p.reshape(NS, NCH, CH)
    src3 = jnp.concatenate([src_t, src_t + N]).reshape(NC * NS, NCH, CH)
    dst3 = dst_p.reshape(NS, NCH, CH)
    zinit = jnp.zeros((ZR, HALF), jnp.float32)

    b_enc2 = b_enc.reshape(1, D)
    b1 = gin_b1.reshape(-1, 1, D)
    b2 = gin_b2.reshape(-1, 1, D)

    hp = _encoder(x, W_enc, b_enc2)
    h0 = jnp.concatenate([hp[0], hp[1]], axis=1)

    for l in range(2):
        agg = _sc_segment_sum(hp.reshape(NC * N, HALF), src3, dst3, zinit)
        hp = _gin_mid(hp, agg.reshape(NC, N, HALF),
                      gin_W1[l], b1[l], gin_W2[l], b2[l])

    agg = _sc_segment_sum(hp.reshape(NC * N, HALF), src3, dst3, zinit)
    h, s = _gin_final(hp, agg.reshape(NC, N, HALF),
                      gin_W1[2], b1[2], gin_W2[2], b2[2])
    return (h, s[0], h0)


# speedup vs baseline: 2.2869x; 1.0389x over previous
import functools

import jax
import jax.numpy as jnp
from jax import lax
from jax.experimental import pallas as pl
from jax.experimental.pallas import tpu as pltpu
from jax.experimental.pallas import tpu_sc as plsc

N = 10000
E = 320000
DIN = 128
D = 256
HALF = 128
NC = 2
NS = 16
CH = 128
NCH = 160
KB = 40
NBLK = NCH // KB
EPAD = NS * NCH * CH
ACC = 10112
ZR = ACC // NS
OPT = 624
TAILO = NS * OPT
TAILN = N - TAILO

BM = 1000
G = N // BM


def _sc_segment_sum(hflat, src3, dst3, zinit):
    @functools.partial(
        pl.kernel,
        out_type=jax.ShapeDtypeStruct((NC * N, HALF), jnp.float32),
        mesh=plsc.VectorSubcoreMesh(core_axis_name="c", subcore_axis_name="s"),
        scratch_types=[
            pltpu.VMEM((KB, CH), jnp.int32),
            pltpu.VMEM((KB, CH), jnp.int32),
            [pltpu.VMEM((CH, HALF), jnp.float32)] * 2,
            pltpu.VMEM_SHARED((ACC, HALF), jnp.float32),
            [pltpu.SemaphoreType.DMA] * 2,
            [pltpu.SemaphoreType.DMA] * 2,
        ],
    )
    def run(h_hbm, s_hbm, d_hbm, z_hbm, out_hbm, s_v, d_v, rows,
            acc, gsem, ssem):
        cid = lax.axis_index("c")
        sid = lax.axis_index("s")
        pltpu.sync_copy(z_hbm, acc.at[pl.ds(sid * ZR, ZR)])
        plsc.subcore_barrier()

        def blk(b, carry):
            pltpu.sync_copy(s_hbm.at[cid * NS + sid, pl.ds(b * KB, KB)], s_v)
            pltpu.sync_copy(d_hbm.at[sid, pl.ds(b * KB, KB)], d_v)
            g = pltpu.async_copy(h_hbm.at[s_v.at[0]], rows[0], gsem[0])
            sc = [None, None]
            for j in range(KB):
                bj = j % 2
                nb = (j + 1) % 2
                if j + 1 < KB:
                    if sc[nb] is not None:
                        sc[nb].wait()
                    gn = pltpu.async_copy(h_hbm.at[s_v.at[j + 1]], rows[nb],
                                          gsem[nb])
                g.wait()
                sc[bj] = pltpu.async_copy(rows[bj], acc.at[d_v.at[j]],
                                          ssem[bj], add=True)
                if j + 1 < KB:
                    g = gn
            sc[0].wait()
            sc[1].wait()
            return carry

        lax.fori_loop(0, NBLK, blk, 0)
        plsc.subcore_barrier()
        pltpu.sync_copy(
            acc.at[pl.ds(sid * OPT, OPT)],
            out_hbm.at[pl.ds(cid * N + sid * OPT, OPT)],
        )

        @pl.when(sid == NS - 1)
        def _():
            pltpu.sync_copy(
                acc.at[pl.ds(TAILO, TAILN)],
                out_hbm.at[pl.ds(cid * N + TAILO, TAILN)],
            )

    return run(hflat, src3, dst3, zinit)


def _relu(v):
    return jnp.maximum(v, 0.0)


def _dot(a, b):
    return jnp.dot(a, b, preferred_element_type=jnp.float32)


def _enc_body(x_ref, w_ref, b_ref, o_ref):
    h = _relu(_dot(x_ref[...], w_ref[...]) + b_ref[...])
    o_ref[0] = h[:, :HALF]
    o_ref[1] = h[:, HALF:]


def _gin_mid_body(h_ref, a_ref, w1_ref, b1_ref, w2_ref, b2_ref, o_ref):
    z0 = h_ref[0] + a_ref[0]
    z1 = h_ref[1] + a_ref[1]
    w1 = w1_ref[...]
    t = _relu(_dot(z0, w1[:HALF]) + _dot(z1, w1[HALF:]) + b1_ref[...])
    u = _relu(_dot(t, w2_ref[...]) + b2_ref[...])
    o_ref[0] = u[:, :HALF]
    o_ref[1] = u[:, HALF:]


def _gin_final_body(h_ref, a_ref, w1_ref, b1_ref, w2_ref, b2_ref, o_ref, s_ref):
    z0 = h_ref[0] + a_ref[0]
    z1 = h_ref[1] + a_ref[1]
    w1 = w1_ref[...]
    t = _relu(_dot(z0, w1[:HALF]) + _dot(z1, w1[HALF:]) + b1_ref[...])
    u = _dot(t, w2_ref[...]) + b2_ref[...]
    o_ref[...] = u
    part = jnp.sum(u, axis=0, keepdims=True)
    i = pl.program_id(0)

    @pl.when(i == 0)
    def _():
        s_ref[...] = part

    @pl.when(i > 0)
    def _():
        s_ref[...] = s_ref[...] + part

    @pl.when(i == G - 1)
    def _():
        s_ref[...] = s_ref[...] * (1.0 / N)


_parts_spec = pl.BlockSpec((NC, BM, HALF), lambda i: (0, i, 0))
_w_spec = pl.BlockSpec((D, D), lambda i: (0, 0))
_b_spec = pl.BlockSpec((1, D), lambda i: (0, 0))


def _encoder(x, w, b):
    return pl.pallas_call(
        _enc_body,
        grid=(G,),
        in_specs=[
            pl.BlockSpec((BM, DIN), lambda i: (i, 0)),
            pl.BlockSpec((DIN, D), lambda i: (0, 0)),
            _b_spec,
        ],
        out_specs=_parts_spec,
        out_shape=jax.ShapeDtypeStruct((NC, N, HALF), jnp.float32),
    )(x, w, b)


def _gin_mid(hp, agg, w1, b1, w2, b2):
    return pl.pallas_call(
        _gin_mid_body,
        grid=(G,),
        in_specs=[_parts_spec, _parts_spec, _w_spec, _b_spec, _w_spec, _b_spec],
        out_specs=_parts_spec,
        out_shape=jax.ShapeDtypeStruct((NC, N, HALF), jnp.float32),
    )(hp, agg, w1, b1, w2, b2)


def _gin_final(hp, agg, w1, b1, w2, b2):
    return pl.pallas_call(
        _gin_final_body,
        grid=(G,),
        in_specs=[_parts_spec, _parts_spec, _w_spec, _b_spec, _w_spec, _b_spec],
        out_specs=[
            pl.BlockSpec((BM, D), lambda i: (i, 0)),
            pl.BlockSpec((1, D), lambda i: (0, 0)),
        ],
        out_shape=[
            jax.ShapeDtypeStruct((N, D), jnp.float32),
            jax.ShapeDtypeStruct((1, D), jnp.float32),
        ],
    )(hp, agg, w1, b1, w2, b2)


def kernel(x, edge_index, W_enc, b_enc, gin_W1, gin_b1, gin_W2, gin_b2):
    src = edge_index[0]
    dst = edge_index[1]
    pad = EPAD - E
    src_p = jnp.concatenate([src, jnp.zeros((pad,), jnp.int32)])
    dst_p = jnp.concatenate([dst, jnp.full((pad,), N, jnp.int32)])
    src_t = src_p.reshape(NS, NCH, CH)
    src3 = jnp.concatenate([src_t, src_t + N]).reshape(NC * NS, NCH, CH)
    dst3 = dst_p.reshape(NS, NCH, CH)
    zinit = jnp.zeros((ZR, HALF), jnp.float32)

    b_enc2 = b_enc.reshape(1, D)
    b1 = gin_b1.reshape(-1, 1, D)
    b2 = gin_b2.reshape(-1, 1, D)

    hp = _encoder(x, W_enc, b_enc2)
    h0 = jnp.concatenate([hp[0], hp[1]], axis=1)

    for l in range(2):
        agg = _sc_segment_sum(hp.reshape(NC * N, HALF), src3, dst3, zinit)
        hp = _gin_mid(hp, agg.reshape(NC, N, HALF),
                      gin_W1[l], b1[l], gin_W2[l], b2[l])

    agg = _sc_segment_sum(hp.reshape(NC * N, HALF), src3, dst3, zinit)
    h, s = _gin_final(hp, agg.reshape(NC, N, HALF),
                      gin_W1[2], b1[2], gin_W2[2], b2[2])
    return (h, s[0], h0)
